# initial kernel scaffold (unmeasured)
import jax
import jax.numpy as jnp
from jax import lax
from jax.experimental import pallas as pl
from jax.experimental.pallas import tpu as pltpu


def kernel(
    x,
):
    def body(*refs):
        pass

    out_shape = jax.ShapeDtypeStruct(..., jnp.float32)
    return pl.pallas_call(body, out_shape=out_shape)(...)


# baseline (device time: 2134626 ns/iter reference)
import jax
import jax.numpy as jnp
from jax import lax
from jax.experimental import pallas as pl
from jax.experimental.pallas import tpu as pltpu


def kernel(x):
    m, n = x.shape

    def body(x_ref, out_ref, send_sem, recv_sem, copy_sem):
        my_x = lax.axis_index("x")
        my_y = lax.axis_index("y")
        my_z = lax.axis_index("z")

        local = pltpu.make_async_copy(
            x_ref, out_ref.at[pl.ds(my_y * m, m)], copy_sem
        )
        local.start()

        rdma = pltpu.make_async_remote_copy(
            src_ref=x_ref,
            dst_ref=out_ref.at[pl.ds(my_y * m, m)],
            send_sem=send_sem,
            recv_sem=recv_sem,
            device_id=(my_x, 1 - my_y, my_z),
            device_id_type=pl.DeviceIdType.MESH,
        )
        rdma.start()
        rdma.wait()
        local.wait()

    return pl.pallas_call(
        body,
        out_shape=jax.ShapeDtypeStruct((2 * m, n), x.dtype),
        in_specs=[pl.BlockSpec(memory_space=pltpu.MemorySpace.HBM)],
        out_specs=pl.BlockSpec(memory_space=pltpu.MemorySpace.HBM),
        scratch_shapes=[
            pltpu.SemaphoreType.DMA,
            pltpu.SemaphoreType.DMA,
            pltpu.SemaphoreType.DMA,
        ],
    )(x)


# device time: 2129531 ns/iter; 1.0024x vs baseline; 1.0024x over previous
import jax
import jax.numpy as jnp
from jax import lax
from jax.experimental import pallas as pl
from jax.experimental.pallas import tpu as pltpu

K = 8


def kernel(x):
    m, n = x.shape
    half_rows = m // 2
    c = half_rows // K

    def body(x_ref, out_ref, y_send, y_recv, f_send, f_recv, copy_sem):
        my_x = lax.axis_index("x")
        my_y = lax.axis_index("y")
        my_z = lax.axis_index("z")
        peer_y = (my_x, 1 - my_y, my_z)
        peer_x = (1 - my_x, my_y, my_z)

        barrier_sem = pltpu.get_barrier_semaphore()
        for nbr in (peer_y, peer_x):
            pl.semaphore_signal(
                barrier_sem, inc=1, device_id=nbr,
                device_id_type=pl.DeviceIdType.MESH,
            )
        pl.semaphore_wait(barrier_sem, 2)

        local = pltpu.make_async_copy(
            x_ref, out_ref.at[pl.ds(my_y * m, m)], copy_sem
        )
        local.start()

        half = my_x * half_rows
        y_rdmas = []
        for k in range(K):
            off = half + k * c
            rd = pltpu.make_async_remote_copy(
                src_ref=x_ref.at[pl.ds(off, c)],
                dst_ref=out_ref.at[pl.ds(my_y * m + off, c)],
                send_sem=y_send.at[k],
                recv_sem=y_recv.at[k],
                device_id=peer_y,
                device_id_type=pl.DeviceIdType.MESH,
            )
            rd.start()
            y_rdmas.append(rd)

        f_rdmas = []
        for k in range(K):
            y_rdmas[k].wait_recv()
            row = (1 - my_y) * m + half + k * c
            fd = pltpu.make_async_remote_copy(
                src_ref=out_ref.at[pl.ds(row, c)],
                dst_ref=out_ref.at[pl.ds(row, c)],
                send_sem=f_send.at[k],
                recv_sem=f_recv.at[k],
                device_id=peer_x,
                device_id_type=pl.DeviceIdType.MESH,
            )
            fd.start()
            f_rdmas.append(fd)

        for k in range(K):
            y_rdmas[k].wait_send()
            f_rdmas[k].wait_send()
            f_rdmas[k].wait_recv()
        local.wait()

    return pl.pallas_call(
        body,
        out_shape=jax.ShapeDtypeStruct((2 * m, n), x.dtype),
        in_specs=[pl.BlockSpec(memory_space=pltpu.MemorySpace.HBM)],
        out_specs=pl.BlockSpec(memory_space=pltpu.MemorySpace.HBM),
        scratch_shapes=[
            pltpu.SemaphoreType.DMA((K,)),
            pltpu.SemaphoreType.DMA((K,)),
            pltpu.SemaphoreType.DMA((K,)),
            pltpu.SemaphoreType.DMA((K,)),
            pltpu.SemaphoreType.DMA,
        ],
        compiler_params=pltpu.CompilerParams(collective_id=0),
    )(x)


# device time: 2127381 ns/iter; 1.0034x vs baseline; 1.0010x over previous
import jax
import jax.numpy as jnp
from jax import lax
from jax.experimental import pallas as pl
from jax.experimental.pallas import tpu as pltpu

K = 16


def kernel(x):
    m, n = x.shape
    c = m // K

    def body(x_ref, out_ref, vbuf, load_sems, send_sems, recv_sems, copy_sem):
        my_x = lax.axis_index("x")
        my_y = lax.axis_index("y")
        my_z = lax.axis_index("z")
        peer_y = (my_x, 1 - my_y, my_z)

        barrier_sem = pltpu.get_barrier_semaphore()
        pl.semaphore_signal(
            barrier_sem, inc=1, device_id=peer_y,
            device_id_type=pl.DeviceIdType.MESH,
        )
        pl.semaphore_wait(barrier_sem, 1)

        local = pltpu.make_async_copy(
            x_ref, out_ref.at[pl.ds(my_y * m, m)], copy_sem
        )
        local.start()

        rdmas = []
        for k in range(K):
            slot = k % 2
            if k >= 2:
                rdmas[k - 2].wait_send()
            load = pltpu.make_async_copy(
                x_ref.at[pl.ds(k * c, c)], vbuf.at[slot], load_sems.at[slot]
            )
            load.start()
            load.wait()
            rd = pltpu.make_async_remote_copy(
                src_ref=vbuf.at[slot],
                dst_ref=out_ref.at[pl.ds(my_y * m + k * c, c)],
                send_sem=send_sems.at[slot],
                recv_sem=recv_sems.at[k],
                device_id=peer_y,
                device_id_type=pl.DeviceIdType.MESH,
            )
            rd.start()
            rdmas.append(rd)

        rdmas[K - 2].wait_send()
        rdmas[K - 1].wait_send()
        for k in range(K):
            rdmas[k].wait_recv()
        local.wait()

    return pl.pallas_call(
        body,
        out_shape=jax.ShapeDtypeStruct((2 * m, n), x.dtype),
        in_specs=[pl.BlockSpec(memory_space=pltpu.MemorySpace.HBM)],
        out_specs=pl.BlockSpec(memory_space=pltpu.MemorySpace.HBM),
        scratch_shapes=[
            pltpu.VMEM((2, c, n), x.dtype),
            pltpu.SemaphoreType.DMA((2,)),
            pltpu.SemaphoreType.DMA((2,)),
            pltpu.SemaphoreType.DMA((K,)),
            pltpu.SemaphoreType.DMA,
        ],
        compiler_params=pltpu.CompilerParams(collective_id=0),
    )(x)
